# initial kernel scaffold (unmeasured)
import jax
import jax.numpy as jnp
from jax import lax
from jax.experimental import pallas as pl
from jax.experimental.pallas import tpu as pltpu


def kernel(
    x,
):
    def body(*refs):
        pass

    out_shape = jax.ShapeDtypeStruct(..., jnp.float32)
    return pl.pallas_call(body, out_shape=out_shape)(...)


# baseline (device time: 565687 ns/iter reference)
import jax
import jax.numpy as jnp
from jax import lax
from jax.experimental import pallas as pl
from jax.experimental.pallas import tpu as pltpu

NZ = 4
M = 4096
K = 4096
CHUNK = 1024
HALF = 512


def kernel(x):
    def body(x_ref, out_ref, comm_p, comm_m, loc_p, loc_m,
             send_sems, recv_sems, loc_sems, out_sems):
        my_x = lax.axis_index("x")
        my_y = lax.axis_index("y")
        my_z = lax.axis_index("z")
        right = (my_z + 1) % NZ
        left = (my_z - 1) % NZ

        barrier = pltpu.get_barrier_semaphore()
        pl.semaphore_signal(barrier, inc=1, device_id=(my_x, my_y, left),
                            device_id_type=pl.DeviceIdType.MESH)
        pl.semaphore_signal(barrier, inc=1, device_id=(my_x, my_y, right),
                            device_id_type=pl.DeviceIdType.MESH)
        pl.semaphore_wait(barrier, 2)

        def rdma(s, plus):
            comm = comm_p if plus else comm_m
            d = 0 if plus else 1
            if s == 0:
                c0 = (my_z - 1) % NZ if plus else (my_z + 1) % NZ
                col = c0 * CHUNK + (0 if plus else HALF)
                src = x_ref.at[0, :, pl.ds(col, HALF)]
            else:
                src = comm.at[(s - 1) % 2]
            tgt = right if plus else left
            return pltpu.make_async_remote_copy(
                src_ref=src,
                dst_ref=comm.at[s % 2],
                send_sem=send_sems.at[d, s],
                recv_sem=recv_sems.at[d, s],
                device_id=(my_x, my_y, tgt),
                device_id_type=pl.DeviceIdType.MESH,
            )

        def stage(s, plus):
            if plus:
                c = (my_z - s - 2) % NZ
                col = c * CHUNK
                cp = pltpu.make_async_copy(
                    x_ref.at[0, :, pl.ds(col, HALF)], loc_p, loc_sems.at[0])
            else:
                c = (my_z + s + 2) % NZ
                col = c * CHUNK + HALF
                cp = pltpu.make_async_copy(
                    x_ref.at[0, :, pl.ds(col, HALF)], loc_m, loc_sems.at[1])
            cp.start()
            return cp

        d_p = rdma(0, True)
        d_m = rdma(0, False)
        d_p.start()
        d_m.start()
        st_p = stage(0, True)
        st_m = stage(0, False)

        for s in range(NZ - 1):
            st_p.wait()
            d_p.wait()
            comm_p[s % 2, :, :] = comm_p[s % 2, :, :] + loc_p[:, :]
            if s < NZ - 2:
                d_p = rdma(s + 1, True)
                d_p.start()
            st_m.wait()
            d_m.wait()
            comm_m[s % 2, :, :] = comm_m[s % 2, :, :] + loc_m[:, :]
            if s < NZ - 2:
                d_m = rdma(s + 1, False)
                d_m.start()
                st_p = stage(s + 1, True)
                st_m = stage(s + 1, False)

        fin = (NZ - 2) % 2
        out_p = pltpu.make_async_copy(
            comm_p.at[fin], out_ref.at[:, pl.ds(0, HALF)], out_sems.at[0])
        out_m = pltpu.make_async_copy(
            comm_m.at[fin], out_ref.at[:, pl.ds(HALF, HALF)], out_sems.at[1])
        out_p.start()
        out_m.start()
        out_p.wait()
        out_m.wait()

    return pl.pallas_call(
        body,
        out_shape=jax.ShapeDtypeStruct((M, CHUNK), jnp.float32),
        in_specs=[pl.BlockSpec(memory_space=pltpu.MemorySpace.HBM)],
        out_specs=pl.BlockSpec(memory_space=pltpu.MemorySpace.HBM),
        scratch_shapes=[
            pltpu.VMEM((2, M, HALF), jnp.float32),
            pltpu.VMEM((2, M, HALF), jnp.float32),
            pltpu.VMEM((M, HALF), jnp.float32),
            pltpu.VMEM((M, HALF), jnp.float32),
            pltpu.SemaphoreType.DMA((2, NZ - 1)),
            pltpu.SemaphoreType.DMA((2, NZ - 1)),
            pltpu.SemaphoreType.DMA((2,)),
            pltpu.SemaphoreType.DMA((2,)),
        ],
        compiler_params=pltpu.CompilerParams(
            collective_id=0, vmem_limit_bytes=56 * 1024 * 1024),
    )(x)


# device time: 222100 ns/iter; 2.5470x vs baseline; 2.5470x over previous
import jax
import jax.numpy as jnp
from jax import lax
from jax.experimental import pallas as pl
from jax.experimental.pallas import tpu as pltpu

NZ = 4
NP = 4
M = 4096
CHUNK = 1024
HALF = 512
RB = 1024


def kernel(x):
    def body(x_ref, out_ref, comm_p, comm_m, loc_p, loc_m, ag_p, ag_m,
             zsend_sems, zrecv_sems, asend_sems, arecv_sems,
             loc_sems, out_sems):
        my_x = lax.axis_index("x")
        my_y = lax.axis_index("y")
        my_z = lax.axis_index("z")
        zright = (my_z + 1) % NZ
        zleft = (my_z - 1) % NZ

        p = 2 * my_x + (my_x + my_y) % 2

        def pos_xy(q):
            return q // 2, (q + q // 2) % 2

        pr_x, pr_y = pos_xy((p + 1) % NP)
        pl_x, pl_y = pos_xy((p - 1) % NP)

        barrier = pltpu.get_barrier_semaphore()
        for dev in ((my_x, my_y, zleft), (my_x, my_y, zright),
                    (pl_x, pl_y, my_z), (pr_x, pr_y, my_z)):
            pl.semaphore_signal(barrier, inc=1, device_id=dev,
                                device_id_type=pl.DeviceIdType.MESH)
        pl.semaphore_wait(barrier, 4)

        row0 = p * RB

        def zrdma(s, plus):
            comm = comm_p if plus else comm_m
            d = 0 if plus else 1
            if s == 0:
                c0 = (my_z - 1) % NZ if plus else (my_z + 1) % NZ
                col = c0 * CHUNK + (0 if plus else HALF)
                src = x_ref.at[0, pl.ds(row0, RB), pl.ds(col, HALF)]
            else:
                src = comm.at[(s - 1) % 2]
            tgt = zright if plus else zleft
            return pltpu.make_async_remote_copy(
                src_ref=src,
                dst_ref=comm.at[s % 2],
                send_sem=zsend_sems.at[d],
                recv_sem=zrecv_sems.at[d, s],
                device_id=(my_x, my_y, tgt),
                device_id_type=pl.DeviceIdType.MESH,
            )

        def stage(s, plus):
            if plus:
                c = (my_z - s - 2) % NZ
                col = c * CHUNK
                cp = pltpu.make_async_copy(
                    x_ref.at[0, pl.ds(row0, RB), pl.ds(col, HALF)],
                    loc_p, loc_sems.at[0])
            else:
                c = (my_z + s + 2) % NZ
                col = c * CHUNK + HALF
                cp = pltpu.make_async_copy(
                    x_ref.at[0, pl.ds(row0, RB), pl.ds(col, HALF)],
                    loc_m, loc_sems.at[1])
            cp.start()
            return cp

        d_p = zrdma(0, True)
        d_m = zrdma(0, False)
        d_p.start()
        d_m.start()
        st_p = stage(0, True)
        st_m = stage(0, False)

        for s in range(NZ - 1):
            st_p.wait()
            d_p.wait()
            comm_p[s % 2, :, :] = comm_p[s % 2, :, :] + loc_p[:, :]
            if s < NZ - 2:
                d_p = zrdma(s + 1, True)
                d_p.start()
            st_m.wait()
            d_m.wait()
            comm_m[s % 2, :, :] = comm_m[s % 2, :, :] + loc_m[:, :]
            if s < NZ - 2:
                d_m = zrdma(s + 1, False)
                d_m.start()
                st_p = stage(s + 1, True)
                st_m = stage(s + 1, False)

        fin = (NZ - 2) % 2

        def agrdma(h, plus):
            ag = ag_p if plus else ag_m
            comm = comm_p if plus else comm_m
            d = 0 if plus else 1
            src = comm.at[fin] if h == 0 else ag.at[h - 1]
            tx, ty = (pr_x, pr_y) if plus else (pl_x, pl_y)
            return pltpu.make_async_remote_copy(
                src_ref=src,
                dst_ref=ag.at[h],
                send_sem=asend_sems.at[d],
                recv_sem=arecv_sems.at[d, h],
                device_id=(tx, ty, my_z),
                device_id_type=pl.DeviceIdType.MESH,
            )

        def store(src_ref, origin, plus, sem_idx):
            col = 0 if plus else HALF
            cp = pltpu.make_async_copy(
                src_ref,
                out_ref.at[pl.ds(origin * RB, RB), pl.ds(col, HALF)],
                out_sems.at[sem_idx])
            cp.start()
            return cp

        a_p = agrdma(0, True)
        a_m = agrdma(0, False)
        a_p.start()
        a_m.start()
        stores = [store(comm_p.at[fin], p, True, 0),
                  store(comm_m.at[fin], p, False, 1)]

        for h in range(NP - 1):
            a_p.wait()
            if h < NP - 2:
                nxt_p = agrdma(h + 1, True)
                nxt_p.start()
            stores.append(store(ag_p.at[h], (p - h - 1) % NP, True, 0))
            a_m.wait()
            if h < NP - 2:
                nxt_m = agrdma(h + 1, False)
                nxt_m.start()
            stores.append(store(ag_m.at[h], (p + h + 1) % NP, False, 1))
            if h < NP - 2:
                a_p, a_m = nxt_p, nxt_m

        for cp in stores:
            cp.wait()

    return pl.pallas_call(
        body,
        out_shape=jax.ShapeDtypeStruct((M, CHUNK), jnp.float32),
        in_specs=[pl.BlockSpec(memory_space=pltpu.MemorySpace.HBM)],
        out_specs=pl.BlockSpec(memory_space=pltpu.MemorySpace.HBM),
        scratch_shapes=[
            pltpu.VMEM((2, RB, HALF), jnp.float32),
            pltpu.VMEM((2, RB, HALF), jnp.float32),
            pltpu.VMEM((RB, HALF), jnp.float32),
            pltpu.VMEM((RB, HALF), jnp.float32),
            pltpu.VMEM((3, RB, HALF), jnp.float32),
            pltpu.VMEM((3, RB, HALF), jnp.float32),
            pltpu.SemaphoreType.DMA((2,)),
            pltpu.SemaphoreType.DMA((2, NZ - 1)),
            pltpu.SemaphoreType.DMA((2,)),
            pltpu.SemaphoreType.DMA((2, NP - 1)),
            pltpu.SemaphoreType.DMA((2,)),
            pltpu.SemaphoreType.DMA((2,)),
        ],
        compiler_params=pltpu.CompilerParams(collective_id=0),
    )(x)
